# baseline (speedup 1.0000x reference)
import functools

import jax
import jax.numpy as jnp
from jax import lax
from jax.experimental import pallas as pl
from jax.experimental.pallas import tpu as pltpu
from jax.experimental.pallas import tpu_sc as plsc

SIGMA = 2.5
NC, NS, LANES = 2, 16, 16
NW = NC * NS
B = 4
KG = 5


def _lane_rot(v, s):
    idx = lax.bitwise_and(lax.iota(jnp.int32, 16) + s, 15)
    return lax.gather(
        v, idx[:, None],
        lax.GatherDimensionNumbers(offset_dims=(), collapsed_slice_dims=(0,),
                                   start_index_map=(0,)),
        (1,), mode=lax.GatherScatterMode.PROMISE_IN_BOUNDS)


def _lane_sum_vec(v):
    for s in (8, 4, 2, 1):
        v = v + _lane_rot(v, s)
    return v


def _bcast(v, lane):
    idx = jnp.zeros((LANES,), jnp.int32) + lane
    return lax.gather(
        v, idx[:, None],
        lax.GatherDimensionNumbers(offset_dims=(), collapsed_slice_dims=(0,),
                                   start_index_map=(0,)),
        (1,), mode=lax.GatherScatterMode.PROMISE_IN_BOUNDS)


def _rsqrt(a, iters=2):
    i = lax.bitcast_convert_type(a, jnp.int32)
    i = jnp.int32(0x5F3759DF) - lax.shift_right_logical(i, 1)
    y = lax.bitcast_convert_type(i, jnp.float32)
    for _ in range(iters):
        y = y * (1.5 - 0.5 * a * y * y)
    return y


def _make_sc_kernel(H, KS, CIN, QW, NB, M_PAD):
    KC = KS * CIN
    NKG = (KS + KG - 1) // KG
    NCH = CIN // LANES
    BH = B * H

    @functools.partial(
        pl.kernel,
        out_type=jax.ShapeDtypeStruct((M_PAD, KC), jnp.float32),
        mesh=plsc.VectorSubcoreMesh(core_axis_name="c", subcore_axis_name="s",
                                    num_cores=NC, num_subcores=NS),
        scratch_types=[
            pltpu.VMEM((NB, BH), jnp.int32),
            pltpu.VMEM((QW, LANES), jnp.float32),
            pltpu.VMEM((4, LANES), jnp.float32),
            pltpu.VMEM((BH, CIN), jnp.float32),
            pltpu.VMEM((BH, CIN), jnp.float32),
            pltpu.VMEM((BH + LANES,), jnp.float32),
            pltpu.VMEM((BH + LANES,), jnp.float32),
            pltpu.VMEM((BH + LANES,), jnp.float32),
            pltpu.VMEM((BH + LANES,), jnp.float32),
            pltpu.VMEM((BH + LANES,), jnp.float32),
            pltpu.VMEM((BH + LANES,), jnp.float32),
            pltpu.VMEM((H + 1, LANES), jnp.float32),
            pltpu.VMEM((B, KC), jnp.float32),
            pltpu.VMEM((B, KC), jnp.float32),
            pltpu.SemaphoreType.DMA,
            pltpu.SemaphoreType.DMA,
            pltpu.SemaphoreType.DMA,
            pltpu.SemaphoreType.DMA,
        ],
    )
    def sc_kernel(feats, px, py, pz, idxf, qp, kpt, out,
                  idx_v, q_v, kpt_v, feats_a, feats_b, xa, ya, za, xb, yb, zb,
                  wbuf, ostage_a, ostage_b, sem_a, sem_b, osem_a, osem_b):
        wid = lax.axis_index("s") * NC + lax.axis_index("c")
        qbase = wid * QW
        pltpu.sync_copy(idxf.at[pl.ds(wid * NB, NB), :], idx_v)
        pltpu.sync_copy(qp.at[pl.ds(qbase, QW), :], q_v)
        pltpu.sync_copy(kpt, kpt_v)

        def fire(bi, fbuf, cx, cy, cz, sem):
            ix = idx_v.at[bi]
            pltpu.async_copy(feats.at[ix], fbuf, sem)
            pltpu.async_copy(px.at[ix], cx.at[pl.ds(0, BH)], sem)
            pltpu.async_copy(py.at[ix], cy.at[pl.ds(0, BH)], sem)
            pltpu.async_copy(pz.at[ix], cz.at[pl.ds(0, BH)], sem)

        def wait(bi, fbuf, cx, cy, cz, sem):
            ix = idx_v.at[bi]
            pltpu.make_async_copy(feats.at[ix], fbuf, sem).wait()
            pltpu.make_async_copy(px.at[ix], cx.at[pl.ds(0, BH)], sem).wait()
            pltpu.make_async_copy(py.at[ix], cy.at[pl.ds(0, BH)], sem).wait()
            pltpu.make_async_copy(pz.at[ix], cz.at[pl.ds(0, BH)], sem).wait()

        kx = kpt_v[0, :]
        ky = kpt_v[1, :]
        kz = kpt_v[2, :]

        def compute_batch(bi, rows, cx, cy, cz, ostage, osem):
            def per_query(qi, _):
                qrow = bi * B + qi
                qv = q_v[qrow, :]
                kxq = kx + _bcast(qv, 0)
                kyq = ky + _bcast(qv, 1)
                kzq = kz + _bcast(qv, 2)
                rbase = qi * H

                def wpass(h, cfv):
                    r = rbase + h
                    dx = _bcast(cx[pl.ds(r, LANES)], 0) - kxq
                    dy = _bcast(cy[pl.ds(r, LANES)], 0) - kyq
                    dz = _bcast(cz[pl.ds(r, LANES)], 0) - kzq
                    d2 = dx * dx + dy * dy + dz * dz
                    a = jnp.maximum(d2, 1e-12)
                    s = a * _rsqrt(a)
                    wbuf[h, :] = jnp.maximum(1.0 - s * (1.0 / SIGMA), 0.0)
                    fs = rows[r, pl.ds(0, LANES)]
                    for j in range(1, NCH):
                        fs = fs + rows[r, pl.ds(LANES * j, LANES)]
                    t = _lane_sum_vec(fs)
                    return cfv + jnp.where(t > 0.0, 1.0, 0.0)

                zero = jnp.zeros((LANES,), jnp.float32)
                cfv = plsc.parallel_loop(0, H, 1, carry=zero)(wpass)
                ry = _rsqrt(jnp.maximum(cfv, 1.0), iters=3)
                invv = ry * ry

                for g in range(NKG):
                    nk = min(KG, KS - g * KG)

                    def apass(h, carry, nk=nk, g=g):
                        accs, ws = carry
                        r = rbase + h
                        wv_n = wbuf[h + 1, :]
                        ws_n = tuple(_bcast(wv_n, g * KG + kk)
                                     for kk in range(nk))
                        new = list(accs)
                        for j in range(NCH):
                            fv = rows[r, pl.ds(LANES * j, LANES)]
                            for kk in range(nk):
                                new[j * nk + kk] = new[j * nk + kk] + ws[kk] * fv
                        return tuple(new), ws_n

                    wv0 = wbuf[0, :]
                    ws0 = tuple(_bcast(wv0, g * KG + kk) for kk in range(nk))
                    accs, _ = plsc.parallel_loop(
                        0, H, 1, carry=((zero,) * (NCH * nk), ws0))(apass)
                    for j in range(NCH):
                        for kk in range(nk):
                            col = (g * KG + kk) * CIN + LANES * j
                            ostage[qi, pl.ds(col, LANES)] = accs[j * nk + kk] * invv
                return 0

            @pl.when(bi >= 2)
            def _():
                pltpu.make_async_copy(
                    ostage, out.at[pl.ds(qbase + (bi - 2) * B, B), :],
                    osem).wait()
            lax.fori_loop(0, B, per_query, 0)
            pltpu.async_copy(ostage, out.at[pl.ds(qbase + bi * B, B), :], osem)

        fire(0, feats_a, xa, ya, za, sem_a)

        def outer(i2, _):
            bi0 = 2 * i2
            fire(bi0 + 1, feats_b, xb, yb, zb, sem_b)
            wait(bi0, feats_a, xa, ya, za, sem_a)
            compute_batch(bi0, feats_a, xa, ya, za, ostage_a, osem_a)

            @pl.when(bi0 + 2 < NB)
            def _():
                fire(bi0 + 2, feats_a, xa, ya, za, sem_a)

            wait(bi0 + 1, feats_b, xb, yb, zb, sem_b)
            compute_batch(bi0 + 1, feats_b, xb, yb, zb, ostage_b, osem_b)
            return 0

        lax.fori_loop(0, NB // 2, outer, 0)
        pltpu.make_async_copy(
            ostage_a, out.at[pl.ds(qbase + (NB - 2) * B, B), :], osem_a).wait()
        pltpu.make_async_copy(
            ostage_b, out.at[pl.ds(qbase + (NB - 1) * B, B), :], osem_b).wait()

    return sc_kernel


def _tc_matmul(wf, w2, m_pad, kc, cout):
    bm = 512

    def mm(wf_ref, w2_ref, o_ref):
        o_ref[...] = jnp.dot(wf_ref[...], w2_ref[...],
                             preferred_element_type=jnp.float32)

    return pl.pallas_call(
        mm,
        grid=(m_pad // bm,),
        in_specs=[
            pl.BlockSpec((bm, kc), lambda i: (i, 0)),
            pl.BlockSpec((kc, cout), lambda i: (0, 0)),
        ],
        out_specs=pl.BlockSpec((bm, cout), lambda i: (i, 0)),
        out_shape=jax.ShapeDtypeStruct((m_pad, cout), jnp.float32),
    )(wf, w2)


def kernel(s_feats, q_points, s_points, neighbor_indices, kernel_points, weights):
    N, CIN = s_feats.shape
    M, H = neighbor_indices.shape
    KS = kernel_points.shape[0]
    COUT = weights.shape[2]
    KC = KS * CIN

    qw = -(-M // NW)
    QW = -(-qw // 8) * 8
    M_PAD = NW * QW
    NB = QW // B

    sf = s_feats.astype(jnp.float32)
    sp = s_points.astype(jnp.float32)
    px, py, pz = sp[:, 0], sp[:, 1], sp[:, 2]

    idx = neighbor_indices.astype(jnp.int32).reshape(M * H)
    idx = jnp.pad(idx, (0, (M_PAD - M) * H)).reshape(NW * NB, B * H)

    qp = jnp.pad(q_points.astype(jnp.float32), ((0, M_PAD - M), (0, 13)))

    kpt = jnp.full((4, LANES), 1e6, jnp.float32)
    kpt = kpt.at[:3, :KS].set(kernel_points.astype(jnp.float32).T)

    sc = _make_sc_kernel(H, KS, CIN, QW, NB, M_PAD)
    wf = sc(sf, px, py, pz, idx, qp, kpt)

    out = _tc_matmul(wf, weights.astype(jnp.float32).reshape(KC, COUT),
                     M_PAD, KC, COUT)
    return out[:M]


# speedup vs baseline: 1.3124x; 1.3124x over previous
import functools

import jax
import jax.numpy as jnp
from jax import lax
from jax.experimental import pallas as pl
from jax.experimental.pallas import tpu as pltpu
from jax.experimental.pallas import tpu_sc as plsc

SIGMA = 2.5
NC, NS, LANES = 2, 16, 16
NW = NC * NS
B = 4
KG = 5


def _lane_rot(v, s):
    idx = lax.bitwise_and(lax.iota(jnp.int32, 16) + s, 15)
    return lax.gather(
        v, idx[:, None],
        lax.GatherDimensionNumbers(offset_dims=(), collapsed_slice_dims=(0,),
                                   start_index_map=(0,)),
        (1,), mode=lax.GatherScatterMode.PROMISE_IN_BOUNDS)


def _lane_sum_vec(v):
    for s in (8, 4, 2, 1):
        v = v + _lane_rot(v, s)
    return v


def _bcast(v, lane):
    idx = jnp.zeros((LANES,), jnp.int32) + lane
    return lax.gather(
        v, idx[:, None],
        lax.GatherDimensionNumbers(offset_dims=(), collapsed_slice_dims=(0,),
                                   start_index_map=(0,)),
        (1,), mode=lax.GatherScatterMode.PROMISE_IN_BOUNDS)


def _rsqrt(a, iters=2):
    i = lax.bitcast_convert_type(a, jnp.int32)
    i = jnp.int32(0x5F3759DF) - lax.shift_right_logical(i, 1)
    y = lax.bitcast_convert_type(i, jnp.float32)
    for _ in range(iters):
        y = y * (1.5 - 0.5 * a * y * y)
    return y


def _make_sc_kernel(H, KS, CIN, QW, NB, M_PAD):
    KC = KS * CIN
    KCP = KC + 128
    NKG = (KS + KG - 1) // KG
    NCH = CIN // LANES
    BH = B * H

    @functools.partial(
        pl.kernel,
        out_type=jax.ShapeDtypeStruct((M_PAD, KCP), jnp.float32),
        mesh=plsc.VectorSubcoreMesh(core_axis_name="c", subcore_axis_name="s",
                                    num_cores=NC, num_subcores=NS),
        scratch_types=[
            pltpu.VMEM((NB, BH), jnp.int32),
            pltpu.VMEM((QW, LANES), jnp.float32),
            pltpu.VMEM((4, LANES), jnp.float32),
            pltpu.VMEM((BH, CIN), jnp.float32),
            pltpu.VMEM((BH, CIN), jnp.float32),
            pltpu.VMEM((BH + LANES,), jnp.float32),
            pltpu.VMEM((BH + LANES,), jnp.float32),
            pltpu.VMEM((BH + LANES,), jnp.float32),
            pltpu.VMEM((BH + LANES,), jnp.float32),
            pltpu.VMEM((BH + LANES,), jnp.float32),
            pltpu.VMEM((BH + LANES,), jnp.float32),
            pltpu.VMEM((BH + LANES,), jnp.float32),
            pltpu.VMEM((BH + LANES,), jnp.float32),
            pltpu.VMEM((H, LANES), jnp.float32),
            pltpu.VMEM((B, KCP), jnp.float32),
            pltpu.VMEM((B, KCP), jnp.float32),
            pltpu.SemaphoreType.DMA,
            pltpu.SemaphoreType.DMA,
            pltpu.SemaphoreType.DMA,
            pltpu.SemaphoreType.DMA,
        ],
    )
    def sc_kernel(feats, px, py, pz, pind, idxf, qp, kpt, out,
                  idx_v, q_v, kpt_v, feats_a, feats_b, xa, ya, za, xb, yb, zb,
                  ia, ib, wbuf, ostage_a, ostage_b,
                  sem_a, sem_b, osem_a, osem_b):
        wid = lax.axis_index("s") * NC + lax.axis_index("c")
        qbase = wid * QW
        pltpu.sync_copy(idxf.at[pl.ds(wid * NB, NB), :], idx_v)
        pltpu.sync_copy(qp.at[pl.ds(qbase, QW), :], q_v)
        pltpu.sync_copy(kpt, kpt_v)

        def fire(bi, fbuf, cx, cy, cz, ci, sem):
            ix = idx_v.at[bi]
            pltpu.async_copy(feats.at[ix], fbuf, sem)
            pltpu.async_copy(px.at[ix], cx.at[pl.ds(0, BH)], sem)
            pltpu.async_copy(py.at[ix], cy.at[pl.ds(0, BH)], sem)
            pltpu.async_copy(pz.at[ix], cz.at[pl.ds(0, BH)], sem)
            pltpu.async_copy(pind.at[ix], ci.at[pl.ds(0, BH)], sem)

        def wait(bi, fbuf, cx, cy, cz, ci, sem):
            ix = idx_v.at[bi]
            pltpu.make_async_copy(feats.at[ix], fbuf, sem).wait()
            pltpu.make_async_copy(px.at[ix], cx.at[pl.ds(0, BH)], sem).wait()
            pltpu.make_async_copy(py.at[ix], cy.at[pl.ds(0, BH)], sem).wait()
            pltpu.make_async_copy(pz.at[ix], cz.at[pl.ds(0, BH)], sem).wait()
            pltpu.make_async_copy(pind.at[ix], ci.at[pl.ds(0, BH)], sem).wait()

        kx = kpt_v[0, :]
        ky = kpt_v[1, :]
        kz = kpt_v[2, :]

        def compute_batch(bi, rows, cx, cy, cz, ci, ostage, osem):
            def per_query(qi, _):
                qrow = bi * B + qi
                qv = q_v[qrow, :]
                kxq = kx + _bcast(qv, 0)
                kyq = ky + _bcast(qv, 1)
                kzq = kz + _bcast(qv, 2)
                rbase = qi * H

                def wpass(h, c):
                    r = rbase + h
                    dx = _bcast(cx[pl.ds(r, LANES)], 0) - kxq
                    dy = _bcast(cy[pl.ds(r, LANES)], 0) - kyq
                    dz = _bcast(cz[pl.ds(r, LANES)], 0) - kzq
                    d2 = dx * dx + dy * dy + dz * dz
                    a = jnp.maximum(d2, 1e-12)
                    s = a * _rsqrt(a)
                    wbuf[h, :] = jnp.maximum(1.0 - s * (1.0 / SIGMA), 0.0)
                    return c

                zero = jnp.zeros((LANES,), jnp.float32)
                plsc.parallel_loop(0, H, 1, carry=jnp.int32(0))(wpass)
                csum = ci[pl.ds(rbase, LANES)]
                for jj in range(1, H // LANES):
                    csum = csum + ci[pl.ds(rbase + LANES * jj, LANES)]
                cfv = _lane_sum_vec(csum)
                ry = _rsqrt(jnp.maximum(cfv, 1.0), iters=3)
                ostage[qi, pl.ds(KC, LANES)] = ry * ry

                for g in range(NKG):
                    nk = min(KG, KS - g * KG)

                    def apass(h, accs, nk=nk, g=g):
                        r = rbase + h
                        wv = wbuf[h, :]
                        ws = [_bcast(wv, g * KG + kk) for kk in range(nk)]
                        new = list(accs)
                        for j in range(NCH):
                            fv = rows[r, pl.ds(LANES * j, LANES)]
                            for kk in range(nk):
                                new[j * nk + kk] = new[j * nk + kk] + ws[kk] * fv
                        return tuple(new)

                    accs = plsc.parallel_loop(
                        0, H, 1, carry=(zero,) * (NCH * nk))(apass)
                    for j in range(NCH):
                        for kk in range(nk):
                            col = (g * KG + kk) * CIN + LANES * j
                            ostage[qi, pl.ds(col, LANES)] = accs[j * nk + kk]
                return 0

            @pl.when(bi >= 2)
            def _():
                pltpu.make_async_copy(
                    ostage, out.at[pl.ds(qbase + (bi - 2) * B, B), :],
                    osem).wait()
            lax.fori_loop(0, B, per_query, 0)
            pltpu.async_copy(ostage, out.at[pl.ds(qbase + bi * B, B), :], osem)

        fire(0, feats_a, xa, ya, za, ia, sem_a)

        def outer(i2, _):
            bi0 = 2 * i2
            fire(bi0 + 1, feats_b, xb, yb, zb, ib, sem_b)
            wait(bi0, feats_a, xa, ya, za, ia, sem_a)
            compute_batch(bi0, feats_a, xa, ya, za, ia, ostage_a, osem_a)

            @pl.when(bi0 + 2 < NB)
            def _():
                fire(bi0 + 2, feats_a, xa, ya, za, ia, sem_a)

            wait(bi0 + 1, feats_b, xb, yb, zb, ib, sem_b)
            compute_batch(bi0 + 1, feats_b, xb, yb, zb, ib, ostage_b, osem_b)
            return 0

        lax.fori_loop(0, NB // 2, outer, 0)
        pltpu.make_async_copy(
            ostage_a, out.at[pl.ds(qbase + (NB - 2) * B, B), :], osem_a).wait()
        pltpu.make_async_copy(
            ostage_b, out.at[pl.ds(qbase + (NB - 1) * B, B), :], osem_b).wait()

    return sc_kernel


def _tc_matmul(wf, w2, m_pad, kc, kcp, cout):
    bm = 512

    def mm(wf_ref, w2_ref, o_ref):
        acc = jnp.dot(wf_ref[:, :kc], w2_ref[...],
                      preferred_element_type=jnp.float32)
        o_ref[...] = acc * wf_ref[:, kc:kc + 1]

    return pl.pallas_call(
        mm,
        grid=(m_pad // bm,),
        in_specs=[
            pl.BlockSpec((bm, kcp), lambda i: (i, 0)),
            pl.BlockSpec((kc, cout), lambda i: (0, 0)),
        ],
        out_specs=pl.BlockSpec((bm, cout), lambda i: (i, 0)),
        out_shape=jax.ShapeDtypeStruct((m_pad, cout), jnp.float32),
    )(wf, w2)


def kernel(s_feats, q_points, s_points, neighbor_indices, kernel_points, weights):
    N, CIN = s_feats.shape
    M, H = neighbor_indices.shape
    KS = kernel_points.shape[0]
    COUT = weights.shape[2]
    KC = KS * CIN

    qw = -(-M // NW)
    QW = -(-qw // 8) * 8
    M_PAD = NW * QW
    NB = QW // B

    sf = s_feats.astype(jnp.float32)
    sp = s_points.astype(jnp.float32)
    px, py, pz = sp[:, 0], sp[:, 1], sp[:, 2]
    pind = (jnp.sum(sf, axis=1) > 0.0).astype(jnp.float32)

    idx = neighbor_indices.astype(jnp.int32).reshape(M * H)
    idx = jnp.pad(idx, (0, (M_PAD - M) * H)).reshape(NW * NB, B * H)

    qp = jnp.pad(q_points.astype(jnp.float32), ((0, M_PAD - M), (0, 13)))

    kpt = jnp.full((4, LANES), 1e6, jnp.float32)
    kpt = kpt.at[:3, :KS].set(kernel_points.astype(jnp.float32).T)

    sc = _make_sc_kernel(H, KS, CIN, QW, NB, M_PAD)
    wf = sc(sf, px, py, pz, pind, idx, qp, kpt)

    out = _tc_matmul(wf, weights.astype(jnp.float32).reshape(KC, COUT),
                     M_PAD, KC, KC + 128, COUT)
    return out[:M]


# speedup vs baseline: 1.3241x; 1.0089x over previous
import functools

import jax
import jax.numpy as jnp
from jax import lax
from jax.experimental import pallas as pl
from jax.experimental.pallas import tpu as pltpu
from jax.experimental.pallas import tpu_sc as plsc

SIGMA = 2.5
NC, NS, LANES = 2, 16, 16
NW = NC * NS
B = 4
KG = 5


def _lane_rot(v, s):
    idx = lax.bitwise_and(lax.iota(jnp.int32, 16) + s, 15)
    return lax.gather(
        v, idx[:, None],
        lax.GatherDimensionNumbers(offset_dims=(), collapsed_slice_dims=(0,),
                                   start_index_map=(0,)),
        (1,), mode=lax.GatherScatterMode.PROMISE_IN_BOUNDS)


def _lane_sum_vec(v):
    for s in (8, 4, 2, 1):
        v = v + _lane_rot(v, s)
    return v


def _bcast(v, lane):
    idx = jnp.zeros((LANES,), jnp.int32) + lane
    return lax.gather(
        v, idx[:, None],
        lax.GatherDimensionNumbers(offset_dims=(), collapsed_slice_dims=(0,),
                                   start_index_map=(0,)),
        (1,), mode=lax.GatherScatterMode.PROMISE_IN_BOUNDS)


def _rsqrt(a, iters=2):
    i = lax.bitcast_convert_type(a, jnp.int32)
    i = jnp.int32(0x5F3759DF) - lax.shift_right_logical(i, 1)
    y = lax.bitcast_convert_type(i, jnp.float32)
    for _ in range(iters):
        y = y * (1.5 - 0.5 * a * y * y)
    return y


def _make_sc_kernel(H, KS, CIN, QW, NB, M_PAD):
    KC = KS * CIN
    KCP = KC + 128
    NKG = (KS + KG - 1) // KG
    NCH = CIN // LANES
    BH = B * H

    @functools.partial(
        pl.kernel,
        out_type=jax.ShapeDtypeStruct((M_PAD, KCP), jnp.float32),
        mesh=plsc.VectorSubcoreMesh(core_axis_name="c", subcore_axis_name="s",
                                    num_cores=NC, num_subcores=NS),
        scratch_types=[
            pltpu.VMEM((NB, BH), jnp.int32),
            pltpu.VMEM((QW, LANES), jnp.float32),
            pltpu.VMEM((4, LANES), jnp.float32),
            pltpu.VMEM((BH, CIN), jnp.float32),
            pltpu.VMEM((BH, CIN), jnp.float32),
            pltpu.VMEM((BH + LANES,), jnp.float32),
            pltpu.VMEM((BH + LANES,), jnp.float32),
            pltpu.VMEM((BH + LANES,), jnp.float32),
            pltpu.VMEM((BH + LANES,), jnp.float32),
            pltpu.VMEM((BH + LANES,), jnp.float32),
            pltpu.VMEM((BH + LANES,), jnp.float32),
            pltpu.VMEM((BH + LANES,), jnp.float32),
            pltpu.VMEM((BH + LANES,), jnp.float32),
            pltpu.VMEM((H, LANES), jnp.float32),
            pltpu.VMEM((B, KCP), jnp.float32),
            pltpu.VMEM((B, KCP), jnp.float32),
            pltpu.SemaphoreType.DMA,
            pltpu.SemaphoreType.DMA,
            pltpu.SemaphoreType.DMA,
            pltpu.SemaphoreType.DMA,
        ],
    )
    def sc_kernel(feats, px, py, pz, pind, idxf, qp, kpt, out,
                  idx_v, q_v, kpt_v, feats_a, feats_b, xa, ya, za, xb, yb, zb,
                  ia, ib, wbuf, ostage_a, ostage_b,
                  sem_a, sem_b, osem_a, osem_b):
        wid = lax.axis_index("s") * NC + lax.axis_index("c")
        qbase = wid * QW
        pltpu.sync_copy(idxf.at[pl.ds(wid * NB, NB), :], idx_v)
        pltpu.sync_copy(qp.at[pl.ds(qbase, QW), :], q_v)
        pltpu.sync_copy(kpt, kpt_v)

        def fire(bi, fbuf, cx, cy, cz, ci, sem):
            ix = idx_v.at[bi]
            pltpu.async_copy(feats.at[ix], fbuf, sem)
            pltpu.async_copy(px.at[ix], cx.at[pl.ds(0, BH)], sem)
            pltpu.async_copy(py.at[ix], cy.at[pl.ds(0, BH)], sem)
            pltpu.async_copy(pz.at[ix], cz.at[pl.ds(0, BH)], sem)
            pltpu.async_copy(pind.at[ix], ci.at[pl.ds(0, BH)], sem)

        def wait(bi, fbuf, cx, cy, cz, ci, sem):
            ix = idx_v.at[bi]
            pltpu.make_async_copy(feats.at[ix], fbuf, sem).wait()
            pltpu.make_async_copy(px.at[ix], cx.at[pl.ds(0, BH)], sem).wait()
            pltpu.make_async_copy(py.at[ix], cy.at[pl.ds(0, BH)], sem).wait()
            pltpu.make_async_copy(pz.at[ix], cz.at[pl.ds(0, BH)], sem).wait()
            pltpu.make_async_copy(pind.at[ix], ci.at[pl.ds(0, BH)], sem).wait()

        kx = kpt_v[0, :]
        ky = kpt_v[1, :]
        kz = kpt_v[2, :]

        def compute_batch(bi, rows, cx, cy, cz, ci, ostage, osem):
            def per_query(qi, _):
                qrow = bi * B + qi
                qv = q_v[qrow, :]
                kxq = kx + _bcast(qv, 0)
                kyq = ky + _bcast(qv, 1)
                kzq = kz + _bcast(qv, 2)
                rbase = qi * H

                def wpass(h, c):
                    for h2 in (h, h + 1):
                        r = rbase + h2
                        dx = _bcast(cx[pl.ds(r, LANES)], 0) - kxq
                        dy = _bcast(cy[pl.ds(r, LANES)], 0) - kyq
                        dz = _bcast(cz[pl.ds(r, LANES)], 0) - kzq
                        d2 = dx * dx + dy * dy + dz * dz
                        a = jnp.maximum(d2, 1e-12)
                        s = a * _rsqrt(a)
                        wbuf[h2, :] = jnp.maximum(1.0 - s * (1.0 / SIGMA), 0.0)
                    return c

                zero = jnp.zeros((LANES,), jnp.float32)
                plsc.parallel_loop(0, H, 2, carry=jnp.int32(0))(wpass)
                csum = ci[pl.ds(rbase, LANES)]
                for jj in range(1, H // LANES):
                    csum = csum + ci[pl.ds(rbase + LANES * jj, LANES)]
                cfv = _lane_sum_vec(csum)
                ry = _rsqrt(jnp.maximum(cfv, 1.0), iters=3)
                ostage[qi, pl.ds(KC, LANES)] = ry * ry

                for g in range(NKG):
                    nk = min(KG, KS - g * KG)

                    def apass(h, accs, nk=nk, g=g):
                        r = rbase + h
                        wv = wbuf[h, :]
                        ws = [_bcast(wv, g * KG + kk) for kk in range(nk)]
                        new = list(accs)
                        for j in range(NCH):
                            fv = rows[r, pl.ds(LANES * j, LANES)]
                            for kk in range(nk):
                                new[j * nk + kk] = new[j * nk + kk] + ws[kk] * fv
                        return tuple(new)

                    accs = plsc.parallel_loop(
                        0, H, 1, carry=(zero,) * (NCH * nk))(apass)
                    for j in range(NCH):
                        for kk in range(nk):
                            col = (g * KG + kk) * CIN + LANES * j
                            ostage[qi, pl.ds(col, LANES)] = accs[j * nk + kk]
                return 0

            @pl.when(bi >= 2)
            def _():
                pltpu.make_async_copy(
                    ostage, out.at[pl.ds(qbase + (bi - 2) * B, B), :],
                    osem).wait()
            lax.fori_loop(0, B, per_query, 0)
            pltpu.async_copy(ostage, out.at[pl.ds(qbase + bi * B, B), :], osem)

        fire(0, feats_a, xa, ya, za, ia, sem_a)

        def outer(i2, _):
            bi0 = 2 * i2
            fire(bi0 + 1, feats_b, xb, yb, zb, ib, sem_b)
            wait(bi0, feats_a, xa, ya, za, ia, sem_a)
            compute_batch(bi0, feats_a, xa, ya, za, ia, ostage_a, osem_a)

            @pl.when(bi0 + 2 < NB)
            def _():
                fire(bi0 + 2, feats_a, xa, ya, za, ia, sem_a)

            wait(bi0 + 1, feats_b, xb, yb, zb, ib, sem_b)
            compute_batch(bi0 + 1, feats_b, xb, yb, zb, ib, ostage_b, osem_b)
            return 0

        lax.fori_loop(0, NB // 2, outer, 0)
        pltpu.make_async_copy(
            ostage_a, out.at[pl.ds(qbase + (NB - 2) * B, B), :], osem_a).wait()
        pltpu.make_async_copy(
            ostage_b, out.at[pl.ds(qbase + (NB - 1) * B, B), :], osem_b).wait()

    return sc_kernel


def _tc_matmul(wf, w2, m_pad, kc, kcp, cout):
    bm = 512

    def mm(wf_ref, w2_ref, o_ref):
        acc = jnp.dot(wf_ref[:, :kc], w2_ref[...],
                      preferred_element_type=jnp.float32)
        o_ref[...] = acc * wf_ref[:, kc:kc + 1]

    return pl.pallas_call(
        mm,
        grid=(m_pad // bm,),
        in_specs=[
            pl.BlockSpec((bm, kcp), lambda i: (i, 0)),
            pl.BlockSpec((kc, cout), lambda i: (0, 0)),
        ],
        out_specs=pl.BlockSpec((bm, cout), lambda i: (i, 0)),
        out_shape=jax.ShapeDtypeStruct((m_pad, cout), jnp.float32),
    )(wf, w2)


def kernel(s_feats, q_points, s_points, neighbor_indices, kernel_points, weights):
    N, CIN = s_feats.shape
    M, H = neighbor_indices.shape
    KS = kernel_points.shape[0]
    COUT = weights.shape[2]
    KC = KS * CIN

    qw = -(-M // NW)
    QW = -(-qw // 8) * 8
    M_PAD = NW * QW
    NB = QW // B

    sf = s_feats.astype(jnp.float32)
    sp = s_points.astype(jnp.float32)
    px, py, pz = sp[:, 0], sp[:, 1], sp[:, 2]
    pind = (jnp.sum(sf, axis=1) > 0.0).astype(jnp.float32)

    idx = neighbor_indices.astype(jnp.int32).reshape(M * H)
    idx = jnp.pad(idx, (0, (M_PAD - M) * H)).reshape(NW * NB, B * H)

    qp = jnp.pad(q_points.astype(jnp.float32), ((0, M_PAD - M), (0, 13)))

    kpt = jnp.full((4, LANES), 1e6, jnp.float32)
    kpt = kpt.at[:3, :KS].set(kernel_points.astype(jnp.float32).T)

    sc = _make_sc_kernel(H, KS, CIN, QW, NB, M_PAD)
    wf = sc(sf, px, py, pz, pind, idx, qp, kpt)

    out = _tc_matmul(wf, weights.astype(jnp.float32).reshape(KC, COUT),
                     M_PAD, KC, KC + 128, COUT)
    return out[:M]
